# Initial kernel scaffold; baseline (speedup 1.0000x reference)
import jax
import jax.numpy as jnp
from jax.experimental import pallas as pl


def kernel(x, tok_emb, pos_emb, ln_g, ln_b, W1, b1, W2, b2):
    raise NotImplementedError("write your pallas kernel here")


# speedup vs baseline: 7.9228x; 7.9228x over previous
import functools

import jax
import jax.numpy as jnp
from jax import lax
from jax.experimental import pallas as pl
from jax.experimental.pallas import tpu as pltpu
from jax.experimental.pallas import tpu_sc as plsc

B, L = 4096, 200
VOCAB, D, HIDDEN = 100000, 128, 64

_NC, _NS = 2, 16
_NW = _NC * _NS
_RPW = B // _NW
_NCHUNK = 2
_CH = L // _NCHUNK
_DV = D // 16


def _sc_gather_sum(x3, tok_emb):
    mesh = plsc.VectorSubcoreMesh(core_axis_name="c", subcore_axis_name="s")

    @functools.partial(
        pl.kernel,
        mesh=mesh,
        out_type=jax.ShapeDtypeStruct((B, D), jnp.float32),
        scratch_types=[
            pltpu.VMEM((_RPW, _NCHUNK, _CH), jnp.int32),
            pltpu.VMEM((L, D), jnp.float32),
            pltpu.VMEM((_RPW, D), jnp.float32),
        ],
    )
    def k(x_hbm, tab_hbm, out_hbm, idx_v, rows_v, out_v):
        wid = lax.axis_index("s") * _NC + lax.axis_index("c")
        base = wid * _RPW
        pltpu.sync_copy(x_hbm.at[pl.ds(base, _RPW)], idx_v)

        def row_body(r, carry):
            for c in range(_NCHUNK):
                pltpu.sync_copy(
                    tab_hbm.at[idx_v.at[r, c]],
                    rows_v.at[pl.ds(c * _CH, _CH)],
                )

            def acc_body(l, accs):
                return tuple(
                    accs[j] + rows_v[l, pl.ds(j * 16, 16)] for j in range(_DV)
                )

            accs = lax.fori_loop(
                0, L, acc_body,
                tuple(jnp.zeros((16,), jnp.float32) for _ in range(_DV)),
            )
            for j in range(_DV):
                out_v[r, pl.ds(j * 16, 16)] = accs[j]
            return carry

        lax.fori_loop(0, _RPW, row_body, 0)
        pltpu.sync_copy(out_v, out_hbm.at[pl.ds(base, _RPW)])

    return k(x3, tok_emb)


def _tc_head(x, sc_sum, tok0, pos_emb, ln_g, ln_b, W1, b1, W2, b2):
    def body(x_ref, s_ref, t0_ref, pos_ref, g_ref, bb_ref,
             w1_ref, b1_ref, w2_ref, b2_ref, o_ref):
        valid = (x_ref[...] != 0).astype(jnp.float32)
        cnt = jnp.sum(valid, axis=1, keepdims=True)
        pooled = (
            s_ref[...]
            - (jnp.float32(L) - cnt) * t0_ref[...]
            + jnp.dot(valid, pos_ref[...], preferred_element_type=jnp.float32)
        ) / jnp.maximum(cnt, 1.0)
        mu = jnp.mean(pooled, axis=1, keepdims=True)
        var = jnp.mean((pooled - mu) ** 2, axis=1, keepdims=True)
        hn = (pooled - mu) / jnp.sqrt(var + 1e-5) * g_ref[...] + bb_ref[...]
        z = jnp.dot(hn, w1_ref[...], preferred_element_type=jnp.float32) + b1_ref[...]
        z = 0.5 * z * (1.0 + lax.erf(z * jnp.float32(0.7071067811865476)))
        score = jnp.dot(z, w2_ref[...], preferred_element_type=jnp.float32) + b2_ref[...]
        o_ref[...] = score - jnp.mean(score)

    return pl.pallas_call(
        body,
        out_shape=jax.ShapeDtypeStruct((B, 1), jnp.float32),
    )(x, sc_sum, tok0, pos_emb, ln_g, ln_b, W1, b1, W2, b2)


def kernel(x, tok_emb, pos_emb, ln_g, ln_b, W1, b1, W2, b2):
    x3 = x.reshape(B, _NCHUNK, _CH)
    sc_sum = _sc_gather_sum(x3, tok_emb)
    score = _tc_head(
        x, sc_sum, tok_emb[0:1], pos_emb,
        ln_g.reshape(1, D), ln_b.reshape(1, D),
        W1, b1.reshape(1, HIDDEN), W2, b2.reshape(1, 1),
    )
    return score[:, 0]


# speedup vs baseline: 16.8100x; 2.1217x over previous
import functools

import jax
import jax.numpy as jnp
from jax import lax
from jax.experimental import pallas as pl
from jax.experimental.pallas import tpu as pltpu
from jax.experimental.pallas import tpu_sc as plsc

B, L = 4096, 200
VOCAB, D, HIDDEN = 100000, 128, 64

_NC, _NS = 2, 16
_NW = _NC * _NS
_RPW = B // _NW
_NCHUNK = 2
_CH = L // _NCHUNK
_DV = D // 16


def _sc_gather_sum(x3, tok_emb):
    mesh = plsc.VectorSubcoreMesh(core_axis_name="c", subcore_axis_name="s")

    @functools.partial(
        pl.kernel,
        mesh=mesh,
        out_type=jax.ShapeDtypeStruct((B, D), jnp.float32),
        scratch_types=[
            pltpu.VMEM((_RPW, _NCHUNK, _CH), jnp.int32),
            pltpu.VMEM((L, D), jnp.float32),
            pltpu.VMEM((L, D), jnp.float32),
            pltpu.VMEM((_RPW, D), jnp.float32),
            pltpu.SemaphoreType.DMA,
            pltpu.SemaphoreType.DMA,
        ],
    )
    def k(x_hbm, tab_hbm, out_hbm, idx_v, buf0, buf1, out_v, sem0, sem1):
        wid = lax.axis_index("s") * _NC + lax.axis_index("c")
        base = wid * _RPW
        pltpu.sync_copy(x_hbm.at[pl.ds(base, _RPW)], idx_v)

        def issue(r, buf, sem):
            for c in range(_NCHUNK):
                pltpu.async_copy(
                    tab_hbm.at[idx_v.at[r, c]],
                    buf.at[pl.ds(c * _CH, _CH)], sem,
                )

        def drain(r, buf, sem):
            for c in range(_NCHUNK):
                pltpu.make_async_copy(
                    tab_hbm.at[idx_v.at[r, c]],
                    buf.at[pl.ds(c * _CH, _CH)], sem,
                ).wait()

        def accum(r, buf):
            def acc_body(l, accs):
                out = []
                for j in range(_DV):
                    a = accs[j] + buf[2 * l, pl.ds(j * 16, 16)]
                    out.append(a + buf[2 * l + 1, pl.ds(j * 16, 16)])
                return tuple(out)

            accs = lax.fori_loop(
                0, L // 2, acc_body,
                tuple(jnp.zeros((16,), jnp.float32) for _ in range(_DV)),
            )
            for j in range(_DV):
                out_v[r, pl.ds(j * 16, 16)] = accs[j]

        issue(0, buf0, sem0)

        def pair_body(r2, carry):
            r0 = 2 * r2
            issue(r0 + 1, buf1, sem1)
            drain(r0, buf0, sem0)
            accum(r0, buf0)

            @pl.when(r2 + 1 < _RPW // 2)
            def _():
                issue(r0 + 2, buf0, sem0)

            drain(r0 + 1, buf1, sem1)
            accum(r0 + 1, buf1)
            return carry

        lax.fori_loop(0, _RPW // 2, pair_body, 0)
        pltpu.sync_copy(out_v, out_hbm.at[pl.ds(base, _RPW)])

    return k(x3, tok_emb)


def _tc_head(x, sc_sum, tok0, pos_emb, ln_g, ln_b, W1, b1, W2, b2):
    def body(x_ref, s_ref, t0_ref, pos_ref, g_ref, bb_ref,
             w1_ref, b1_ref, w2_ref, b2_ref, o_ref):
        valid = (x_ref[...] != 0).astype(jnp.float32)
        cnt = jnp.sum(valid, axis=1, keepdims=True)
        pooled = (
            s_ref[...]
            - (jnp.float32(L) - cnt) * t0_ref[...]
            + jnp.dot(valid, pos_ref[...], preferred_element_type=jnp.float32)
        ) / jnp.maximum(cnt, 1.0)
        mu = jnp.mean(pooled, axis=1, keepdims=True)
        var = jnp.mean((pooled - mu) ** 2, axis=1, keepdims=True)
        hn = (pooled - mu) / jnp.sqrt(var + 1e-5) * g_ref[...] + bb_ref[...]
        z = jnp.dot(hn, w1_ref[...], preferred_element_type=jnp.float32) + b1_ref[...]
        z = 0.5 * z * (1.0 + lax.erf(z * jnp.float32(0.7071067811865476)))
        score = jnp.dot(z, w2_ref[...], preferred_element_type=jnp.float32) + b2_ref[...]
        o_ref[...] = score - jnp.mean(score)

    return pl.pallas_call(
        body,
        out_shape=jax.ShapeDtypeStruct((B, 1), jnp.float32),
    )(x, sc_sum, tok0, pos_emb, ln_g, ln_b, W1, b1, W2, b2)


def kernel(x, tok_emb, pos_emb, ln_g, ln_b, W1, b1, W2, b2):
    x3 = x.reshape(B, _NCHUNK, _CH)
    sc_sum = _sc_gather_sum(x3, tok_emb)
    score = _tc_head(
        x, sc_sum, tok_emb[0:1], pos_emb,
        ln_g.reshape(1, D), ln_b.reshape(1, D),
        W1, b1.reshape(1, HIDDEN), W2, b2.reshape(1, 1),
    )
    return score[:, 0]


# speedup vs baseline: 16.8428x; 1.0019x over previous
import functools

import jax
import jax.numpy as jnp
from jax import lax
from jax.experimental import pallas as pl
from jax.experimental.pallas import tpu as pltpu
from jax.experimental.pallas import tpu_sc as plsc

B, L = 4096, 200
VOCAB, D, HIDDEN = 100000, 128, 64

_NC, _NS = 2, 16
_NW = _NC * _NS
_RPW = B // _NW
_NCHUNK = 2
_CH = L // _NCHUNK
_DV = D // 16


def _sc_gather_sum(x3, tok_emb):
    mesh = plsc.VectorSubcoreMesh(core_axis_name="c", subcore_axis_name="s")

    @functools.partial(
        pl.kernel,
        mesh=mesh,
        out_type=jax.ShapeDtypeStruct((B, D), jnp.float32),
        scratch_types=[
            pltpu.VMEM((_RPW, _NCHUNK, _CH), jnp.int32),
            pltpu.VMEM((L, D), jnp.float32),
            pltpu.VMEM((L, D), jnp.float32),
            pltpu.VMEM((_RPW, D), jnp.float32),
            pltpu.SemaphoreType.DMA,
            pltpu.SemaphoreType.DMA,
        ],
    )
    def k(x_hbm, tab_hbm, out_hbm, idx_v, buf0, buf1, out_v, sem0, sem1):
        wid = lax.axis_index("s") * _NC + lax.axis_index("c")
        base = wid * _RPW
        pltpu.sync_copy(x_hbm.at[pl.ds(base, _RPW)], idx_v)

        def issue(r, buf, sem):
            for c in range(_NCHUNK):
                pltpu.async_copy(
                    tab_hbm.at[idx_v.at[r, c]],
                    buf.at[pl.ds(c * _CH, _CH)], sem,
                )

        def drain(r, buf, sem):
            for c in range(_NCHUNK):
                pltpu.make_async_copy(
                    tab_hbm.at[idx_v.at[r, c]],
                    buf.at[pl.ds(c * _CH, _CH)], sem,
                ).wait()

        def accum(r, buf):
            init = tuple(jnp.zeros((16,), jnp.float32) for _ in range(_DV))

            @plsc.parallel_loop(0, L // 2, unroll=4, carry=init)
            def accs(l, accs):
                out = []
                for j in range(_DV):
                    a = accs[j] + buf[2 * l, pl.ds(j * 16, 16)]
                    out.append(a + buf[2 * l + 1, pl.ds(j * 16, 16)])
                return tuple(out)

            for j in range(_DV):
                out_v[r, pl.ds(j * 16, 16)] = accs[j]

        issue(0, buf0, sem0)

        def pair_body(r2, carry):
            r0 = 2 * r2
            issue(r0 + 1, buf1, sem1)
            drain(r0, buf0, sem0)
            accum(r0, buf0)

            @pl.when(r2 + 1 < _RPW // 2)
            def _():
                issue(r0 + 2, buf0, sem0)

            drain(r0 + 1, buf1, sem1)
            accum(r0 + 1, buf1)
            return carry

        lax.fori_loop(0, _RPW // 2, pair_body, 0)
        pltpu.sync_copy(out_v, out_hbm.at[pl.ds(base, _RPW)])

    return k(x3, tok_emb)


def _tc_head(x, sc_sum, tok0, pos_emb, ln_g, ln_b, W1, b1, W2, b2):
    def body(x_ref, s_ref, t0_ref, pos_ref, g_ref, bb_ref,
             w1_ref, b1_ref, w2_ref, b2_ref, o_ref):
        valid = (x_ref[...] != 0).astype(jnp.float32)
        cnt = jnp.sum(valid, axis=1, keepdims=True)
        pooled = (
            s_ref[...]
            - (jnp.float32(L) - cnt) * t0_ref[...]
            + jnp.dot(valid, pos_ref[...], preferred_element_type=jnp.float32)
        ) / jnp.maximum(cnt, 1.0)
        mu = jnp.mean(pooled, axis=1, keepdims=True)
        var = jnp.mean((pooled - mu) ** 2, axis=1, keepdims=True)
        hn = (pooled - mu) / jnp.sqrt(var + 1e-5) * g_ref[...] + bb_ref[...]
        z = jnp.dot(hn, w1_ref[...], preferred_element_type=jnp.float32) + b1_ref[...]
        z = 0.5 * z * (1.0 + lax.erf(z * jnp.float32(0.7071067811865476)))
        score = jnp.dot(z, w2_ref[...], preferred_element_type=jnp.float32) + b2_ref[...]
        o_ref[...] = score - jnp.mean(score)

    return pl.pallas_call(
        body,
        out_shape=jax.ShapeDtypeStruct((B, 1), jnp.float32),
    )(x, sc_sum, tok0, pos_emb, ln_g, ln_b, W1, b1, W2, b2)


def kernel(x, tok_emb, pos_emb, ln_g, ln_b, W1, b1, W2, b2):
    x3 = x.reshape(B, _NCHUNK, _CH)
    sc_sum = _sc_gather_sum(x3, tok_emb)
    score = _tc_head(
        x, sc_sum, tok_emb[0:1], pos_emb,
        ln_g.reshape(1, D), ln_b.reshape(1, D),
        W1, b1.reshape(1, HIDDEN), W2, b2.reshape(1, 1),
    )
    return score[:, 0]


# speedup vs baseline: 20.3948x; 1.2109x over previous
import functools

import jax
import jax.numpy as jnp
from jax import lax
from jax.experimental import pallas as pl
from jax.experimental.pallas import tpu as pltpu
from jax.experimental.pallas import tpu_sc as plsc

B, L = 4096, 200
VOCAB, D, HIDDEN = 100000, 128, 64

_NC, _NS = 2, 16
_NW = _NC * _NS
_RPW = B // _NW
_NCHUNK = 2
_CH = L // _NCHUNK
_DV = D // 16


def _sc_gather_sum(x3, tok_emb):
    mesh = plsc.VectorSubcoreMesh(core_axis_name="c", subcore_axis_name="s")

    @functools.partial(
        pl.kernel,
        mesh=mesh,
        out_type=jax.ShapeDtypeStruct((B, D), jnp.float32),
        scratch_types=[
            pltpu.VMEM((_RPW, _NCHUNK, _CH), jnp.int32),
            pltpu.VMEM((_CH, D), jnp.float32),
            pltpu.VMEM((_CH, D), jnp.float32),
            pltpu.VMEM((_CH, D), jnp.float32),
            pltpu.VMEM((_CH, D), jnp.float32),
            pltpu.VMEM((_RPW, D), jnp.float32),
            pltpu.SemaphoreType.DMA,
            pltpu.SemaphoreType.DMA,
            pltpu.SemaphoreType.DMA,
            pltpu.SemaphoreType.DMA,
        ],
    )
    def k(x_hbm, tab_hbm, out_hbm, idx_v,
          buf0, buf1, buf2, buf3, out_v, sem0, sem1, sem2, sem3):
        wid = lax.axis_index("s") * _NC + lax.axis_index("c")
        base = wid * _RPW
        pltpu.sync_copy(x_hbm.at[pl.ds(base, _RPW)], idx_v)

        bufs = (buf0, buf1, buf2, buf3)
        sems = (sem0, sem1, sem2, sem3)

        def issue(r, c, buf, sem):
            pltpu.async_copy(tab_hbm.at[idx_v.at[r, c]], buf, sem)

        def drain(r, c, buf, sem):
            pltpu.make_async_copy(tab_hbm.at[idx_v.at[r, c]], buf, sem).wait()

        def accum_chunk(buf, accs):
            @plsc.parallel_loop(0, _CH // 2, unroll=4, carry=accs)
            def out(l, accs):
                res = []
                for j in range(_DV):
                    a = accs[j] + buf[2 * l, pl.ds(j * 16, 16)]
                    res.append(a + buf[2 * l + 1, pl.ds(j * 16, 16)])
                return tuple(res)
            return out

        for m in range(4):
            issue(m // 2, m % 2, bufs[m], sems[m])

        _K = _RPW // 2

        def body(k2, carry):
            r0 = 2 * k2
            zero = tuple(jnp.zeros((16,), jnp.float32) for _ in range(_DV))
            for half in range(2):
                r = r0 + half
                accs = zero
                for c in range(_NCHUNK):
                    m = 2 * half + c
                    drain(r, c, bufs[m], sems[m])
                    accs = accum_chunk(bufs[m], accs)

                    @pl.when(k2 + 1 < _K)
                    def _():
                        issue(r + 2, c, bufs[m], sems[m])

                for j in range(_DV):
                    out_v[r, pl.ds(j * 16, 16)] = accs[j]
            return carry

        lax.fori_loop(0, _K, body, 0)
        pltpu.sync_copy(out_v, out_hbm.at[pl.ds(base, _RPW)])

    return k(x3, tok_emb)


def _tc_head(x, sc_sum, tok0, pos_emb, ln_g, ln_b, W1, b1, W2, b2):
    def body(x_ref, s_ref, t0_ref, pos_ref, g_ref, bb_ref,
             w1_ref, b1_ref, w2_ref, b2_ref, o_ref):
        valid = (x_ref[...] != 0).astype(jnp.float32)
        cnt = jnp.sum(valid, axis=1, keepdims=True)
        pooled = (
            s_ref[...]
            - (jnp.float32(L) - cnt) * t0_ref[...]
            + jnp.dot(valid, pos_ref[...], preferred_element_type=jnp.float32)
        ) / jnp.maximum(cnt, 1.0)
        mu = jnp.mean(pooled, axis=1, keepdims=True)
        var = jnp.mean((pooled - mu) ** 2, axis=1, keepdims=True)
        hn = (pooled - mu) / jnp.sqrt(var + 1e-5) * g_ref[...] + bb_ref[...]
        z = jnp.dot(hn, w1_ref[...], preferred_element_type=jnp.float32) + b1_ref[...]
        z = 0.5 * z * (1.0 + lax.erf(z * jnp.float32(0.7071067811865476)))
        score = jnp.dot(z, w2_ref[...], preferred_element_type=jnp.float32) + b2_ref[...]
        o_ref[...] = score - jnp.mean(score)

    return pl.pallas_call(
        body,
        out_shape=jax.ShapeDtypeStruct((B, 1), jnp.float32),
    )(x, sc_sum, tok0, pos_emb, ln_g, ln_b, W1, b1, W2, b2)


def kernel(x, tok_emb, pos_emb, ln_g, ln_b, W1, b1, W2, b2):
    x3 = x.reshape(B, _NCHUNK, _CH)
    sc_sum = _sc_gather_sum(x3, tok_emb)
    score = _tc_head(
        x, sc_sum, tok_emb[0:1], pos_emb,
        ln_g.reshape(1, D), ln_b.reshape(1, D),
        W1, b1.reshape(1, HIDDEN), W2, b2.reshape(1, 1),
    )
    return score[:, 0]
